# baseline (speedup 1.0000x reference)
import functools

import jax
import jax.numpy as jnp
from jax import lax
from jax.experimental import pallas as pl
from jax.experimental.pallas import tpu as pltpu
from jax.experimental.pallas import tpu_sc as plsc

_FROZEN = 0
_LANES = 16


@functools.lru_cache(maxsize=None)
def _build_sc_kernel(n_tokens, vocab, r, feat, n_workers, chunk):
    assert n_tokens % (n_workers * chunk) == 0
    assert r & (r - 1) == 0 and r <= _LANES
    assert feat % _LANES == 0
    tok_per_w = n_tokens // n_workers
    n_chunks = tok_per_w // chunk
    assert n_chunks % 2 == 0 and chunk % 2 == 0
    cr = chunk * r
    f_groups = feat // _LANES
    rows_per_line = _LANES // r
    assert vocab % rows_per_line == 0
    r_shift = r.bit_length() - 1
    line_shift = rows_per_line.bit_length() - 1

    mesh = plsc.VectorSubcoreMesh(core_axis_name="c", subcore_axis_name="s")

    @functools.partial(
        pl.kernel,
        out_type=jax.ShapeDtypeStruct((n_tokens, feat), jnp.bfloat16),
        mesh=mesh,
        scratch_types=[
            pltpu.VMEM((tok_per_w,), jnp.int32),
            pltpu.VMEM((tok_per_w,), jnp.int32),
            pltpu.VMEM((2, chunk, _LANES), jnp.int32),
            pltpu.VMEM((2, cr), jnp.int32),
            pltpu.VMEM((2, chunk, _LANES), jnp.int32),
            pltpu.VMEM((2, cr, feat), jnp.bfloat16),
            pltpu.VMEM((2, chunk, feat), jnp.bfloat16),
            pltpu.VMEM_SHARED((vocab // rows_per_line, _LANES), jnp.int32),
            pltpu.SemaphoreType.DMA,
            [pltpu.SemaphoreType.DMA] * 2,
            [pltpu.SemaphoreType.DMA] * 2,
            [pltpu.SemaphoreType.DMA] * 2,
        ],
        compiler_params=pltpu.CompilerParams(
            use_tc_tiling_on_sc=False, needs_layout_passes=False),
    )
    def sc_kernel(bidx_hbm, res_hbm, emb_hbm, out_hbm,
                  bidx_v, line_v, rline_v, ridx_v, mask_v, erows_v, out_v,
                  res_sh, bsem, lsems, esems, osems):
        n_cores = mesh.num_cores
        wid = lax.axis_index("s") * n_cores + lax.axis_index("c")
        base = pl.multiple_of(wid * tok_per_w, chunk)

        def tok0_of(cidx):
            return pl.multiple_of(base + cidx * chunk, chunk)

        n_lines = vocab // rows_per_line
        span = n_lines // mesh.num_subcores
        rem = n_lines - span * mesh.num_subcores
        sid = lax.axis_index("s")
        lo = sid * span
        pltpu.sync_copy(res_hbm.at[pl.ds(lo, span)], res_sh.at[pl.ds(lo, span)])
        if rem:
            @pl.when(sid == 0)
            def _():
                pltpu.sync_copy(
                    res_hbm.at[pl.ds(span * mesh.num_subcores, rem)],
                    res_sh.at[pl.ds(span * mesh.num_subcores, rem)])

        pltpu.async_copy(
            bidx_hbm.at[pl.ds(base, tok_per_w)], bidx_v, bsem).wait()

        @pl.loop(0, tok_per_w // _LANES)
        def _line_loop(i):
            bv = bidx_v[pl.ds(_LANES * i, _LANES)]
            line_v[pl.ds(_LANES * i, _LANES)] = bv >> line_shift

        plsc.subcore_barrier()

        def line_slice(cidx):
            return line_v.at[pl.ds(pl.multiple_of(cidx * chunk, chunk), chunk)]

        def stage_b(b, cidx):
            pltpu.async_copy(
                res_sh.at[line_slice(cidx)], rline_v.at[b], lsems[b])

        def stage_c(b, cidx):
            pltpu.make_async_copy(
                res_sh.at[line_slice(cidx)], rline_v.at[b], lsems[b]).wait()
            rb, ib, mb = rline_v.at[b], ridx_v.at[b], mask_v.at[b]
            cbase = cidx * chunk

            @pl.loop(0, cr // _LANES)
            def _build_loop(j):
                ones = jnp.full((_LANES,), 0x3F803F80, dtype=jnp.int32)
                zeros = jnp.zeros((_LANES,), dtype=jnp.int32)
                iota = lax.iota(jnp.int32, _LANES)
                lane_r = iota & (r - 1)
                tok = (iota >> r_shift) + rows_per_line * j
                bvals = plsc.load_gather(bidx_v, [cbase + tok])
                off = ((bvals & (rows_per_line - 1)) << r_shift) + lane_r
                vals = plsc.load_gather(rb, [tok, off])
                ib[pl.ds(_LANES * j, _LANES)] = vals
                plsc.store_scatter(
                    mb, [tok, lane_r],
                    jnp.where(vals != _FROZEN, ones, zeros))

            pltpu.async_copy(emb_hbm.at[ib], erows_v.at[b], esems[b])

        def stage_d(b, cidx):
            eb, mb, ob = erows_v.at[b], mask_v.at[b], out_v.at[b]
            pltpu.make_async_copy(
                emb_hbm.at[ridx_v.at[b]], eb, esems[b]).wait()

            @pl.when(cidx >= 2)
            def _():
                pltpu.make_async_copy(
                    ob, out_hbm.at[pl.ds(tok0_of(cidx - 2), chunk)],
                    osems[b]).wait()

            @pl.loop(0, chunk // 2)
            def _accum_loop(t2):
                for half in range(2):
                    t = 2 * t2 + half
                    row0 = r * t
                    mv = mb[t, pl.ds(0, _LANES)]
                    masks = [
                        plsc.bitcast(
                            lax.broadcast(mv[i], (_LANES,)), jnp.bfloat16)
                        for i in range(r)]
                    for k in range(f_groups // 2):
                        s = pl.ds(2 * _LANES * k, 2 * _LANES)
                        parts = [eb[row0 + i, s] * masks[i] for i in range(r)]
                        while len(parts) > 1:
                            parts = [parts[i] + parts[i + 1]
                                     for i in range(0, len(parts) - 1, 2)] + (
                                         [parts[-1]] if len(parts) % 2 else [])
                        ob[t, s] = parts[0]

            pltpu.async_copy(
                ob, out_hbm.at[pl.ds(tok0_of(cidx), chunk)], osems[b])

        stage_b(0, 0)
        stage_c(0, 0)
        stage_b(1, 1)

        @pl.loop(0, n_chunks, step=2)
        def _chunk_loop(g):
            stage_c(1, g + 1)
            stage_d(0, g)

            @pl.when(g + 2 < n_chunks)
            def _():
                stage_b(0, g + 2)
                stage_c(0, g + 2)

            stage_d(1, g + 1)

            @pl.when(g + 3 < n_chunks)
            def _():
                stage_b(1, g + 3)

        pltpu.make_async_copy(
            out_v.at[0], out_hbm.at[pl.ds(tok0_of(n_chunks - 2), chunk)],
            osems[0]).wait()
        pltpu.make_async_copy(
            out_v.at[1], out_hbm.at[pl.ds(tok0_of(n_chunks - 1), chunk)],
            osems[1]).wait()

    return sc_kernel


def kernel(base_indices, reservoir_encoded, embedding_weight):
    b, l = base_indices.shape
    vocab, r = reservoir_encoded.shape
    feat = embedding_weight.shape[1]
    n_tokens = b * l
    flat = base_indices.reshape(n_tokens)
    rows_per_line = _LANES // r
    res_lines = reservoir_encoded.reshape(vocab // rows_per_line, _LANES)
    sc = _build_sc_kernel(n_tokens, vocab, r, feat, n_workers=32, chunk=128)
    out = sc(flat, res_lines, embedding_weight.astype(jnp.bfloat16))
    return out.astype(jnp.float32).reshape(b, l, feat)


# speedup vs baseline: 1.1180x; 1.1180x over previous
import functools

import jax
import jax.numpy as jnp
from jax import lax
from jax.experimental import pallas as pl
from jax.experimental.pallas import tpu as pltpu
from jax.experimental.pallas import tpu_sc as plsc

_FROZEN = 0
_LANES = 16


@functools.lru_cache(maxsize=None)
def _build_sc_kernel(n_tokens, vocab, r, feat, n_workers, chunk):
    assert n_tokens % (n_workers * chunk) == 0
    assert r & (r - 1) == 0 and r <= _LANES
    assert feat % _LANES == 0
    tok_per_w = n_tokens // n_workers
    n_chunks = tok_per_w // chunk
    assert n_chunks % 2 == 0 and chunk % 2 == 0
    cr = chunk * r
    f_groups = feat // _LANES
    rows_per_line = _LANES // r
    assert vocab % rows_per_line == 0
    r_shift = r.bit_length() - 1
    line_shift = rows_per_line.bit_length() - 1

    mesh = plsc.VectorSubcoreMesh(core_axis_name="c", subcore_axis_name="s")

    @functools.partial(
        pl.kernel,
        out_type=jax.ShapeDtypeStruct((n_tokens, feat), jnp.float32),
        mesh=mesh,
        scratch_types=[
            pltpu.VMEM((tok_per_w,), jnp.int32),
            pltpu.VMEM((tok_per_w,), jnp.int32),
            pltpu.VMEM((2, chunk, _LANES), jnp.int32),
            pltpu.VMEM((2, cr), jnp.int32),
            pltpu.VMEM((2, chunk, _LANES), jnp.float32),
            pltpu.VMEM((2, cr, feat), jnp.float32),
            pltpu.VMEM((2, chunk, feat), jnp.float32),
            pltpu.VMEM_SHARED((vocab // rows_per_line, _LANES), jnp.int32),
            pltpu.SemaphoreType.DMA,
            [pltpu.SemaphoreType.DMA] * 2,
            [pltpu.SemaphoreType.DMA] * 2,
            [pltpu.SemaphoreType.DMA] * 2,
        ],
        compiler_params=pltpu.CompilerParams(
            use_tc_tiling_on_sc=False, needs_layout_passes=False),
    )
    def sc_kernel(bidx_hbm, res_hbm, emb_hbm, out_hbm,
                  bidx_v, line_v, rline_v, ridx_v, mask_v, erows_v, out_v,
                  res_sh, bsem, lsems, esems, osems):
        n_cores = mesh.num_cores
        wid = lax.axis_index("s") * n_cores + lax.axis_index("c")
        base = pl.multiple_of(wid * tok_per_w, chunk)

        def tok0_of(cidx):
            return pl.multiple_of(base + cidx * chunk, chunk)

        n_lines = vocab // rows_per_line
        span = n_lines // mesh.num_subcores
        rem = n_lines - span * mesh.num_subcores
        sid = lax.axis_index("s")
        lo = sid * span
        pltpu.sync_copy(res_hbm.at[pl.ds(lo, span)], res_sh.at[pl.ds(lo, span)])
        if rem:
            @pl.when(sid == 0)
            def _():
                pltpu.sync_copy(
                    res_hbm.at[pl.ds(span * mesh.num_subcores, rem)],
                    res_sh.at[pl.ds(span * mesh.num_subcores, rem)])

        pltpu.async_copy(
            bidx_hbm.at[pl.ds(base, tok_per_w)], bidx_v, bsem).wait()

        @pl.loop(0, tok_per_w // _LANES)
        def _line_loop(i):
            bv = bidx_v[pl.ds(_LANES * i, _LANES)]
            line_v[pl.ds(_LANES * i, _LANES)] = bv >> line_shift

        plsc.subcore_barrier()

        def line_slice(cidx):
            return line_v.at[pl.ds(pl.multiple_of(cidx * chunk, chunk), chunk)]

        def stage_b(b, cidx):
            pltpu.async_copy(
                res_sh.at[line_slice(cidx)], rline_v.at[b], lsems[b])

        def stage_c(b, cidx):
            pltpu.make_async_copy(
                res_sh.at[line_slice(cidx)], rline_v.at[b], lsems[b]).wait()
            rb, ib, mb = rline_v.at[b], ridx_v.at[b], mask_v.at[b]
            cbase = cidx * chunk

            @pl.loop(0, cr // _LANES)
            def _build_loop(j):
                ones = jnp.full((_LANES,), 1.0, dtype=jnp.float32)
                zeros = jnp.zeros((_LANES,), dtype=jnp.float32)
                iota = lax.iota(jnp.int32, _LANES)
                lane_r = iota & (r - 1)
                tok = (iota >> r_shift) + rows_per_line * j
                bvals = plsc.load_gather(bidx_v, [cbase + tok])
                off = ((bvals & (rows_per_line - 1)) << r_shift) + lane_r
                vals = plsc.load_gather(rb, [tok, off])
                ib[pl.ds(_LANES * j, _LANES)] = vals
                plsc.store_scatter(
                    mb, [tok, lane_r],
                    jnp.where(vals != _FROZEN, ones, zeros))

            pltpu.async_copy(emb_hbm.at[ib], erows_v.at[b], esems[b])

        def stage_d(b, cidx):
            eb, mb, ob = erows_v.at[b], mask_v.at[b], out_v.at[b]
            pltpu.make_async_copy(
                emb_hbm.at[ridx_v.at[b]], eb, esems[b]).wait()

            @pl.when(cidx >= 2)
            def _():
                pltpu.make_async_copy(
                    ob, out_hbm.at[pl.ds(tok0_of(cidx - 2), chunk)],
                    osems[b]).wait()

            @pl.loop(0, chunk // 2)
            def _accum_loop(t2):
                for half in range(2):
                    t = 2 * t2 + half
                    row0 = r * t
                    mv = mb[t, pl.ds(0, _LANES)]
                    for k in range(f_groups):
                        s = pl.ds(_LANES * k, _LANES)
                        parts = [eb[row0 + i, s] * mv[i] for i in range(r)]
                        while len(parts) > 1:
                            parts = [parts[i] + parts[i + 1]
                                     for i in range(0, len(parts) - 1, 2)] + (
                                         [parts[-1]] if len(parts) % 2 else [])
                        ob[t, s] = parts[0]

            pltpu.async_copy(
                ob, out_hbm.at[pl.ds(tok0_of(cidx), chunk)], osems[b])

        stage_b(0, 0)
        stage_c(0, 0)
        stage_b(1, 1)

        @pl.loop(0, n_chunks, step=2)
        def _chunk_loop(g):
            stage_c(1, g + 1)
            stage_d(0, g)

            @pl.when(g + 2 < n_chunks)
            def _():
                stage_b(0, g + 2)
                stage_c(0, g + 2)

            stage_d(1, g + 1)

            @pl.when(g + 3 < n_chunks)
            def _():
                stage_b(1, g + 3)

        pltpu.make_async_copy(
            out_v.at[0], out_hbm.at[pl.ds(tok0_of(n_chunks - 2), chunk)],
            osems[0]).wait()
        pltpu.make_async_copy(
            out_v.at[1], out_hbm.at[pl.ds(tok0_of(n_chunks - 1), chunk)],
            osems[1]).wait()

    return sc_kernel


def kernel(base_indices, reservoir_encoded, embedding_weight):
    b, l = base_indices.shape
    vocab, r = reservoir_encoded.shape
    feat = embedding_weight.shape[1]
    n_tokens = b * l
    flat = base_indices.reshape(n_tokens)
    rows_per_line = _LANES // r
    res_lines = reservoir_encoded.reshape(vocab // rows_per_line, _LANES)
    sc = _build_sc_kernel(n_tokens, vocab, r, feat, n_workers=32, chunk=128)
    out = sc(flat, res_lines, embedding_weight)
    return out.reshape(b, l, feat)
